# Initial kernel scaffold; baseline (speedup 1.0000x reference)
import jax
import jax.numpy as jnp
from jax.experimental import pallas as pl


def kernel(fea, fea_adj, adj, global_weight, W1, b1, W2, b2, W3, b3, Wd1, bd1, Wd2, bd2):
    raise NotImplementedError("write your pallas kernel here")


# speedup vs baseline: 1.4938x; 1.4938x over previous
import jax
import jax.numpy as jnp
from jax.experimental import pallas as pl
from jax.experimental.pallas import tpu as pltpu


def _bce(logits, targets):
    return (jnp.maximum(logits, 0.0) - logits * targets
            + jnp.log1p(jnp.exp(-jnp.abs(logits))))


def _enc1_body(fea_ref, W1_ref, b1_ref, adj_ref, x_ref, s1_ref):
    @pl.when(pl.program_id(0) == 0)
    def _():
        s1_ref[...] = jnp.dot(fea_ref[...], W1_ref[...],
                              preferred_element_type=jnp.float32)
    x_ref[...] = jax.nn.sigmoid(
        jnp.dot(adj_ref[...], s1_ref[...],
                preferred_element_type=jnp.float32) + b1_ref[...])


def _enc2_body(x_ref, W23_ref, b23_ref, adj_ref, eps_ref, fea_ref,
               Wd1T_ref, bd1_ref, Wd2T_ref, bd2_ref,
               z_ref, acc_ref, s23_ref):
    i = pl.program_id(0)

    @pl.when(i == 0)
    def _():
        s23_ref[...] = jnp.dot(x_ref[...], W23_ref[...],
                               preferred_element_type=jnp.float32)
        acc_ref[...] = jnp.zeros_like(acc_ref)

    E = eps_ref.shape[1]
    ml = jnp.dot(adj_ref[...], s23_ref[...],
                 preferred_element_type=jnp.float32) + b23_ref[...]
    mu = ml[:, :E]
    logvar = ml[:, E:]
    std = jnp.exp(0.5 * logvar)
    ev = std * std
    z = eps_ref[...] * std + mu
    z_ref[...] = z
    kld = -0.5 * jnp.sum(1.0 + logvar - mu * mu - ev)
    h = jax.nn.sigmoid(jnp.dot(z, Wd1T_ref[...],
                               preferred_element_type=jnp.float32)
                       + bd1_ref[...])
    recon = jnp.dot(h, Wd2T_ref[...],
                    preferred_element_type=jnp.float32) + bd2_ref[...]
    fb = jnp.sum(_bce(recon, fea_ref[...]))
    acc_ref[...] += (kld + fb).reshape(1, 1)


def _adj_bce_body(bm, bn, z_ref, gw_ref, adj_ref, acc_ref):
    i = pl.program_id(0)
    j = pl.program_id(1)

    @pl.when((i == 0) & (j == 0))
    def _():
        acc_ref[...] = jnp.zeros_like(acc_ref)

    zi = z_ref[pl.ds(i * bm, bm), :]
    zj = z_ref[pl.ds(j * bn, bn), :]
    r = jax.lax.dot_general(zi, zj, (((1,), (1,)), ((), ())),
                            preferred_element_type=jnp.float32)
    a = adj_ref[...]
    v = jnp.maximum(r, 0.0) - r * a + jnp.log1p(jnp.exp(-jnp.abs(r)))
    acc_ref[...] += gw_ref[...] * jnp.sum(a * v)


def kernel(fea, fea_adj, adj, global_weight, W1, b1, W2, b2, W3, b3,
           Wd1, bd1, Wd2, bd2):
    del fea_adj
    N, F = fea.shape
    R = W1.shape[1]
    E = W2.shape[1]

    BM = 512
    nI = N // BM

    b1r = b1.reshape(1, R)
    W23 = jnp.concatenate([W2, W3], axis=1)
    b23 = jnp.concatenate([b2, b3]).reshape(1, 2 * E)
    Wd1T = Wd1.T
    bd1r = bd1.reshape(1, R)
    Wd2T = Wd2.T
    bd2r = bd2.reshape(1, F)
    eps = jax.random.normal(jax.random.key(42), (N, E), dtype=jnp.float32)
    gw = global_weight.reshape(1, 1)

    x = pl.pallas_call(
        _enc1_body,
        grid=(nI,),
        in_specs=[
            pl.BlockSpec((N, F), lambda i: (0, 0)),
            pl.BlockSpec((F, R), lambda i: (0, 0)),
            pl.BlockSpec((1, R), lambda i: (0, 0)),
            pl.BlockSpec((BM, N), lambda i: (i, 0)),
        ],
        out_specs=pl.BlockSpec((BM, R), lambda i: (i, 0)),
        out_shape=jax.ShapeDtypeStruct((N, R), jnp.float32),
        scratch_shapes=[pltpu.VMEM((N, R), jnp.float32)],
        compiler_params=pltpu.CompilerParams(
            dimension_semantics=("arbitrary",)),
    )(fea, W1, b1r, adj)

    z, acc1 = pl.pallas_call(
        _enc2_body,
        grid=(nI,),
        in_specs=[
            pl.BlockSpec((N, R), lambda i: (0, 0)),
            pl.BlockSpec((R, 2 * E), lambda i: (0, 0)),
            pl.BlockSpec((1, 2 * E), lambda i: (0, 0)),
            pl.BlockSpec((BM, N), lambda i: (i, 0)),
            pl.BlockSpec((BM, E), lambda i: (i, 0)),
            pl.BlockSpec((BM, F), lambda i: (i, 0)),
            pl.BlockSpec((E, R), lambda i: (0, 0)),
            pl.BlockSpec((1, R), lambda i: (0, 0)),
            pl.BlockSpec((R, F), lambda i: (0, 0)),
            pl.BlockSpec((1, F), lambda i: (0, 0)),
        ],
        out_specs=[
            pl.BlockSpec((BM, E), lambda i: (i, 0)),
            pl.BlockSpec((1, 1), lambda i: (0, 0)),
        ],
        out_shape=[
            jax.ShapeDtypeStruct((N, E), jnp.float32),
            jax.ShapeDtypeStruct((1, 1), jnp.float32),
        ],
        scratch_shapes=[pltpu.VMEM((N, 2 * E), jnp.float32)],
        compiler_params=pltpu.CompilerParams(
            dimension_semantics=("arbitrary",)),
    )(x, W23, b23, adj, eps, fea, Wd1T, bd1r, Wd2T, bd2r)

    BM2, BN = 512, 512
    nI2, nJ = N // BM2, N // BN
    acc3 = pl.pallas_call(
        lambda *refs: _adj_bce_body(BM2, BN, *refs),
        grid=(nI2, nJ),
        in_specs=[
            pl.BlockSpec((N, E), lambda i, j: (0, 0)),
            pl.BlockSpec((1, 1), lambda i, j: (0, 0)),
            pl.BlockSpec((BM2, BN), lambda i, j: (i, j)),
        ],
        out_specs=pl.BlockSpec((1, 1), lambda i, j: (0, 0)),
        out_shape=jax.ShapeDtypeStruct((1, 1), jnp.float32),
        compiler_params=pltpu.CompilerParams(
            dimension_semantics=("arbitrary", "arbitrary")),
    )(z, gw, adj)

    return acc1[0, 0] + acc3[0, 0]
